# baseline (speedup 1.0000x reference)
import functools

import jax
import jax.numpy as jnp
import numpy as np
from jax import lax
from jax.experimental import pallas as pl
from jax.experimental.pallas import tpu as pltpu
from jax.experimental.pallas import tpu_sc as plsc

N = 10000
E = 320000
D = 128

NC = 2
NS = 16
NW = NC * NS
EW = E // NW
CH = 100
NPT = 624
NTAIL = N - NS * NPT
CPT = EW // CH


def _edge_logits(eap, wf1, wf2, wf3):
    Eb = 2000

    def body(ea_ref, w1_ref, w2_ref, w3_ref, o1_ref, o2_ref, o3_ref):
        a = ea_ref[...]
        o1_ref[...] = jnp.dot(a, w1_ref[...], preferred_element_type=jnp.float32)
        o2_ref[...] = jnp.dot(a, w2_ref[...], preferred_element_type=jnp.float32)
        o3_ref[...] = jnp.dot(a, w3_ref[...], preferred_element_type=jnp.float32)

    wspec = pl.BlockSpec((128, 128), lambda i: (0, 0))
    espec = pl.BlockSpec((Eb, 128), lambda i: (i, 0))
    out = pl.pallas_call(
        body,
        grid=(E // 8 // Eb,),
        in_specs=[espec, wspec, wspec, wspec],
        out_specs=[espec] * 3,
        out_shape=[jax.ShapeDtypeStruct((E // 8, 128), jnp.float32)] * 3,
    )(eap, wf1, wf2, wf3)
    return out


def _node_dense(x, w, ams, amd, prev=None):
    Nb = 1000
    wspec = pl.BlockSpec((128, 128), lambda i: (0, 0))
    aspec = pl.BlockSpec((128, 16), lambda i: (0, 0))
    nspec = pl.BlockSpec((Nb, 128), lambda i: (i, 0))
    dspec = pl.BlockSpec((Nb, 16), lambda i: (i, 0))
    n2spec = pl.BlockSpec((Nb, 128), lambda i: (i + N // Nb, 0))
    d2spec = pl.BlockSpec((Nb, 16), lambda i: (i + N // Nb, 0))

    if prev is None:
        def body(x_ref, w_ref, ams_ref, amd_ref, h_ref, als_ref, ald_ref):
            h = jnp.dot(x_ref[...], w_ref[...], preferred_element_type=jnp.float32)
            h_ref[...] = h
            als_ref[...] = jnp.dot(h, ams_ref[...], preferred_element_type=jnp.float32)
            ald_ref[...] = jnp.dot(h, amd_ref[...], preferred_element_type=jnp.float32)

        in_specs = [nspec, wspec, aspec, aspec]
        args = (x, w, ams, amd)
    else:
        num, den, expand, bias = prev

        def body(n0_ref, n1_ref, d0_ref, d1_ref, ex_ref, b_ref, w_ref,
                 ams_ref, amd_ref, h_ref, als_ref, ald_ref):
            den = d0_ref[...] + d1_ref[...]
            de = jnp.dot(den, ex_ref[...], preferred_element_type=jnp.float32)
            xv = (n0_ref[...] + n1_ref[...]) / (de + 1e-16) + b_ref[...]
            xv = jnp.maximum(xv, 0.0)
            h = jnp.dot(xv, w_ref[...], preferred_element_type=jnp.float32)
            h_ref[...] = h
            als_ref[...] = jnp.dot(h, ams_ref[...], preferred_element_type=jnp.float32)
            ald_ref[...] = jnp.dot(h, amd_ref[...], preferred_element_type=jnp.float32)

        in_specs = [nspec, n2spec, dspec, d2spec,
                    pl.BlockSpec((16, 128), lambda i: (0, 0)),
                    pl.BlockSpec((1, 128), lambda i: (0, 0)),
                    wspec, aspec, aspec]
        args = (num, num, den, den, expand, bias, w, ams, amd)

    return pl.pallas_call(
        body,
        grid=(N // Nb,),
        in_specs=in_specs,
        out_specs=[nspec, dspec, dspec],
        out_shape=[jax.ShapeDtypeStruct((N, 128), jnp.float32),
                   jax.ShapeDtypeStruct((N, 16), jnp.float32),
                   jax.ShapeDtypeStruct((N, 16), jnp.float32)],
    )(*args)


def _final_dense(num, den, expand, bias, wl, bl):
    Nb = 1000

    def body(n0_ref, n1_ref, d0_ref, d1_ref, ex_ref, b_ref, w_ref, bl_ref, o_ref):
        den = d0_ref[...] + d1_ref[...]
        de = jnp.dot(den, ex_ref[...], preferred_element_type=jnp.float32)
        xv = (n0_ref[...] + n1_ref[...]) / (de + 1e-16) + b_ref[...]
        xv = jnp.maximum(xv, 0.0)
        o = jnp.dot(xv, w_ref[...], preferred_element_type=jnp.float32) + bl_ref[...]
        o_ref[...] = jnp.maximum(o, 0.0)

    nspec = pl.BlockSpec((Nb, 128), lambda i: (i, 0))
    dspec = pl.BlockSpec((Nb, 16), lambda i: (i, 0))
    n2spec = pl.BlockSpec((Nb, 128), lambda i: (i + N // Nb, 0))
    d2spec = pl.BlockSpec((Nb, 16), lambda i: (i + N // Nb, 0))
    return pl.pallas_call(
        body,
        grid=(N // Nb,),
        in_specs=[nspec, n2spec, dspec, d2spec,
                  pl.BlockSpec((16, 128), lambda i: (0, 0)),
                  pl.BlockSpec((1, 128), lambda i: (0, 0)),
                  pl.BlockSpec((128, 128), lambda i: (0, 0)),
                  pl.BlockSpec((1, 128), lambda i: (0, 0))],
        out_specs=nspec,
        out_shape=jax.ShapeDtypeStruct((N, 128), jnp.float32),
    )(num, num, den, den, expand, bias, wl, bl)


def _make_edge_pass(n_heads):
    head_of = [j if n_heads == 8 else 0 for j in range(8)]
    mesh = plsc.VectorSubcoreMesh(core_axis_name="c", subcore_axis_name="s")

    @functools.partial(
        pl.kernel,
        mesh=mesh,
        compiler_params=pltpu.CompilerParams(use_tc_tiling_on_sc=False),
        out_type=[jax.ShapeDtypeStruct((NC * N, 128), jnp.float32),
                  jax.ShapeDtypeStruct((NC * N, 16), jnp.float32)],
        scratch_types=[
            pltpu.VMEM_SHARED((N, 128), jnp.float32),
            pltpu.VMEM_SHARED((N, 16), jnp.float32),
            pltpu.VMEM((CH,), jnp.int32),
            pltpu.VMEM((CH,), jnp.int32),
            pltpu.VMEM((CH,), jnp.int32),
            pltpu.VMEM((CH,), jnp.int32),
            pltpu.VMEM((CH,), jnp.int32),
            pltpu.VMEM((CH,), jnp.int32),
            pltpu.VMEM((CH,), jnp.int32),
            pltpu.VMEM((CH,), jnp.int32),
            pltpu.VMEM((CH, 128), jnp.float32),
            pltpu.VMEM((CH, 128), jnp.float32),
            pltpu.VMEM((CH, 16), jnp.float32),
            pltpu.VMEM((CH, 16), jnp.float32),
            pltpu.VMEM((CH, 16), jnp.float32),
            pltpu.VMEM((CH, 16), jnp.float32),
            pltpu.VMEM((CH, 16), jnp.float32),
            pltpu.VMEM((CH, 16), jnp.float32),
            pltpu.VMEM((CH, 16), jnp.float32),
            pltpu.VMEM((CH, 16), jnp.float32),
        ] + [pltpu.SemaphoreType.DMA] * 6,
    )
    def edge_pass(src_hbm, dst_hbm, h_hbm, als_hbm, ald_hbm, ale_hbm,
                  num_out, den_out,
                  num_sh, den_sh,
                  srcI0, srcI1, srcI2, srcI3, dstI0, dstI1, dstI2, dstI3,
                  hg0, hg1, ale0, ale1, als0, als1, ald0, ald1, wb0, wb1,
                  bsem0, bsem1, gsem0, gsem1, ssem0, ssem1):
        c = lax.axis_index("c")
        s = lax.axis_index("s")
        wid = c * NS + s
        srcI = [srcI0, srcI1, srcI2, srcI3]
        dstI = [dstI0, dstI1, dstI2, dstI3]
        hg = [hg0, hg1]
        alel = [ale0, ale1]
        alsg = [als0, als1]
        aldg = [ald0, ald1]
        wb = [wb0, wb1]
        bsem = [bsem0, bsem1]
        gsem = [gsem0, gsem1]
        ssem = [ssem0, ssem1]

        def base_copies(g, r, p):
            row = wid * CPT + g
            return [
                pltpu.make_async_copy(src_hbm.at[row], srcI[r], bsem[p]),
                pltpu.make_async_copy(dst_hbm.at[row], dstI[r], bsem[p]),
                pltpu.make_async_copy(ale_hbm.at[row], alel[p], bsem[p]),
            ]

        def gather_copies(p, r):
            return [
                pltpu.make_async_copy(als_hbm.at[srcI[r]], alsg[p], gsem[p]),
                pltpu.make_async_copy(ald_hbm.at[dstI[r]], aldg[p], gsem[p]),
                pltpu.make_async_copy(h_hbm.at[srcI[r]], hg[p], gsem[p]),
            ]

        def scatter_copies(p, r):
            return [
                pltpu.make_async_copy(hg[p], num_sh.at[dstI[r]], ssem[p]),
                pltpu.make_async_copy(wb[p], den_sh.at[dstI[r]], ssem[p]),
            ]

        def fire_scatters(p, r):
            pltpu.async_copy(hg[p], num_sh.at[dstI[r]], ssem[p], add=True)
            pltpu.async_copy(wb[p], den_sh.at[dstI[r]], ssem[p], add=True)

        zv = jnp.zeros((16,), jnp.float32)

        def zrow(r, _):
            for j in range(8):
                hg0[r, pl.ds(j * 16, 16)] = zv
            wb0[r, :] = zv
            return 0

        lax.fori_loop(0, CH, zrow, 0)
        r0 = s * NPT
        done = 0
        while done < NPT:
            step = min(CH, NPT - done)
            pltpu.sync_copy(hg0.at[pl.ds(0, step)], num_sh.at[pl.ds(r0 + done, step)])
            pltpu.sync_copy(wb0.at[pl.ds(0, step)], den_sh.at[pl.ds(r0 + done, step)])
            done += step

        @pl.when(s == 0)
        def _zero_tail():
            pltpu.sync_copy(hg0.at[pl.ds(0, NTAIL)], num_sh.at[pl.ds(NS * NPT, NTAIL)])
            pltpu.sync_copy(wb0.at[pl.ds(0, NTAIL)], den_sh.at[pl.ds(NS * NPT, NTAIL)])

        plsc.subcore_barrier()

        for cp in base_copies(0, 0, 0) + base_copies(1, 1, 1):
            cp.start()
        for cp in base_copies(0, 0, 0):
            cp.wait()
        for cp in gather_copies(0, 0):
            cp.start()

        def half(g, b):
            p = b % 2
            q = 1 - p
            rn = (b + 1) % 4
            rp = (b - 1) % 4
            rf = (b + 2) % 4

            @pl.when(g + 1 < CPT)
            def _advance_other():
                for cp in base_copies(g + 1, rn, q):
                    cp.wait()

                @pl.when(g >= 1)
                def _drain_scatter_q():
                    for cp in scatter_copies(q, rp):
                        cp.wait()

                for cp in gather_copies(q, rn):
                    cp.start()

            for cp in gather_copies(p, b):
                cp.wait()

            def wpass(e, _):
                a = alsg[p][e, :] + aldg[p][e, :] + alel[p][e, :]
                a = jnp.where(a >= 0.0, a, 0.2 * a)
                wb[p][e, :] = jnp.exp(a)
                return 0

            def mpass(e, _):
                w = wb[p][e, :]
                for j in range(8):
                    hg[p][e, pl.ds(j * 16, 16)] = (
                        hg[p][e, pl.ds(j * 16, 16)] * w[head_of[j]])
                return 0

            lax.fori_loop(0, CH, wpass, 0, unroll=2)
            lax.fori_loop(0, CH, mpass, 0, unroll=2)
            fire_scatters(p, b)

            @pl.when(g + 2 < CPT)
            def _prefetch_base():
                for cp in base_copies(g + 2, rf, p):
                    cp.start()

        def pipe(i, _):
            for b in range(4):
                half(4 * i + b, b)
            return 0

        lax.fori_loop(0, CPT // 4, pipe, 0)
        for cp in scatter_copies(0, 2) + scatter_copies(1, 3):
            cp.wait()
        plsc.subcore_barrier()

        pltpu.sync_copy(num_sh.at[pl.ds(r0, NPT)], num_out.at[pl.ds(c * N + r0, NPT)])
        pltpu.sync_copy(den_sh.at[pl.ds(r0, NPT)], den_out.at[pl.ds(c * N + r0, NPT)])

        @pl.when(s == 0)
        def _write_tail():
            pltpu.sync_copy(num_sh.at[pl.ds(NS * NPT, NTAIL)],
                            num_out.at[pl.ds(c * N + NS * NPT, NTAIL)])
            pltpu.sync_copy(den_sh.at[pl.ds(NS * NPT, NTAIL)],
                            den_out.at[pl.ds(c * N + NS * NPT, NTAIL)])

    return edge_pass


_edge_pass_h8 = _make_edge_pass(8)
_edge_pass_h1 = _make_edge_pass(1)


def _fold_we(We, a_e, H, C):
    wf = (We.reshape(We.shape[0], H, C) * a_e[None]).sum(-1)
    return jnp.pad(wf, ((0, 0), (0, 16 - H)))

def _a_mat(a, H, C):
    m = jnp.zeros((H * C, 16), jnp.float32)
    for h in range(H):
        m = m.at[h * C:(h + 1) * C, h].set(a[h])
    return m

def _expand_mat(H):
    m = np.zeros((16, 128), np.float32)
    C = 128 // H
    for h in range(H):
        m[h, h * C:(h + 1) * C] = 1.0
    return jnp.asarray(m)


def kernel(x, edge_index, edge_attr, W1, We1, as1, ad1, ae1, b1,
           W2, We2, as2, ad2, ae2, b2, W3, We3, as3, ad3, ae3, b3, Wl, bl):
    src = edge_index[0].reshape(E // CH, CH)
    dst = edge_index[1].reshape(E // CH, CH)

    eap = edge_attr.reshape(E // 8, 128)
    eye8 = jnp.eye(8, dtype=jnp.float32)
    ale1, ale2, ale3 = _edge_logits(
        eap, jnp.kron(eye8, _fold_we(We1, ae1, 8, 16)),
        jnp.kron(eye8, _fold_we(We2, ae2, 8, 16)),
        jnp.kron(eye8, _fold_we(We3, ae3, 1, 128)))
    ale1 = ale1.reshape(E // CH, CH, 16)
    ale2 = ale2.reshape(E // CH, CH, 16)
    ale3 = ale3.reshape(E // CH, CH, 16)

    ex8 = _expand_mat(8)
    ex1 = _expand_mat(1)

    h, als, ald = _node_dense(x, W1, _a_mat(as1, 8, 16), _a_mat(ad1, 8, 16))
    num, den = _edge_pass_h8(src, dst, h, als, ald, ale1)

    h, als, ald = _node_dense(
        None, W2, _a_mat(as2, 8, 16), _a_mat(ad2, 8, 16),
        prev=(num, den, ex8, b1.reshape(1, 128)))
    num, den = _edge_pass_h8(src, dst, h, als, ald, ale2)

    h, als, ald = _node_dense(
        None, W3, _a_mat(as3, 1, 128), _a_mat(ad3, 1, 128),
        prev=(num, den, ex8, b2.reshape(1, 128)))
    num, den = _edge_pass_h1(src, dst, h, als, ald, ale3)

    return _final_dense(num, den, ex1,
                        b3.reshape(1, 128), Wl, bl.reshape(1, 128))


# speedup vs baseline: 1.3146x; 1.3146x over previous
import functools

import jax
import jax.numpy as jnp
import numpy as np
from jax import lax
from jax.experimental import pallas as pl
from jax.experimental.pallas import tpu as pltpu
from jax.experimental.pallas import tpu_sc as plsc

N = 10000
E = 320000
D = 128

NC = 2
NS = 16
NW = NC * NS
EW = E // NW
CH = 100
NPT = 624
NTAIL = N - NS * NPT
CPT = EW // CH


def _edge_logits(eap, wf1, wf2, wf3):
    Eb = 2000

    def body(ea_ref, w1_ref, w2_ref, w3_ref, o1_ref, o2_ref, o3_ref):
        a = ea_ref[...]
        o1_ref[...] = jnp.dot(a, w1_ref[...], preferred_element_type=jnp.float32)
        o2_ref[...] = jnp.dot(a, w2_ref[...], preferred_element_type=jnp.float32)
        o3_ref[...] = jnp.dot(a, w3_ref[...], preferred_element_type=jnp.float32)

    wspec = pl.BlockSpec((128, 128), lambda i: (0, 0))
    espec = pl.BlockSpec((Eb, 128), lambda i: (i, 0))
    out = pl.pallas_call(
        body,
        grid=(E // 8 // Eb,),
        in_specs=[espec, wspec, wspec, wspec],
        out_specs=[espec] * 3,
        out_shape=[jax.ShapeDtypeStruct((E // 8, 128), jnp.float32)] * 3,
    )(eap, wf1, wf2, wf3)
    return out


def _node_dense(x, w, ams, amd, prev=None):
    Nb = 1000
    wspec = pl.BlockSpec((128, 128), lambda i: (0, 0))
    aspec = pl.BlockSpec((128, 16), lambda i: (0, 0))
    nspec = pl.BlockSpec((Nb, 128), lambda i: (i, 0))
    dspec = pl.BlockSpec((Nb, 16), lambda i: (i, 0))
    n2spec = pl.BlockSpec((Nb, 128), lambda i: (i + N // Nb, 0))
    d2spec = pl.BlockSpec((Nb, 16), lambda i: (i + N // Nb, 0))

    if prev is None:
        def body(x_ref, w_ref, ams_ref, amd_ref, h_ref, als_ref, ald_ref):
            h = jnp.dot(x_ref[...], w_ref[...], preferred_element_type=jnp.float32)
            h_ref[...] = h
            als_ref[...] = jnp.dot(h, ams_ref[...], preferred_element_type=jnp.float32)
            ald_ref[...] = jnp.dot(h, amd_ref[...], preferred_element_type=jnp.float32)

        in_specs = [nspec, wspec, aspec, aspec]
        args = (x, w, ams, amd)
    else:
        num, den, expand, bias = prev

        def body(n0_ref, n1_ref, d0_ref, d1_ref, ex_ref, b_ref, w_ref,
                 ams_ref, amd_ref, h_ref, als_ref, ald_ref):
            den = d0_ref[...] + d1_ref[...]
            de = jnp.dot(den, ex_ref[...], preferred_element_type=jnp.float32)
            xv = (n0_ref[...] + n1_ref[...]) / (de + 1e-16) + b_ref[...]
            xv = jnp.maximum(xv, 0.0)
            h = jnp.dot(xv, w_ref[...], preferred_element_type=jnp.float32)
            h_ref[...] = h
            als_ref[...] = jnp.dot(h, ams_ref[...], preferred_element_type=jnp.float32)
            ald_ref[...] = jnp.dot(h, amd_ref[...], preferred_element_type=jnp.float32)

        in_specs = [nspec, n2spec, dspec, d2spec,
                    pl.BlockSpec((16, 128), lambda i: (0, 0)),
                    pl.BlockSpec((1, 128), lambda i: (0, 0)),
                    wspec, aspec, aspec]
        args = (num, num, den, den, expand, bias, w, ams, amd)

    return pl.pallas_call(
        body,
        grid=(N // Nb,),
        in_specs=in_specs,
        out_specs=[nspec, dspec, dspec],
        out_shape=[jax.ShapeDtypeStruct((N, 128), jnp.float32),
                   jax.ShapeDtypeStruct((N, 16), jnp.float32),
                   jax.ShapeDtypeStruct((N, 16), jnp.float32)],
    )(*args)


def _final_dense(num, den, expand, bias, wl, bl):
    Nb = 1000

    def body(n0_ref, n1_ref, d0_ref, d1_ref, ex_ref, b_ref, w_ref, bl_ref, o_ref):
        den = d0_ref[...] + d1_ref[...]
        de = jnp.dot(den, ex_ref[...], preferred_element_type=jnp.float32)
        xv = (n0_ref[...] + n1_ref[...]) / (de + 1e-16) + b_ref[...]
        xv = jnp.maximum(xv, 0.0)
        o = jnp.dot(xv, w_ref[...], preferred_element_type=jnp.float32) + bl_ref[...]
        o_ref[...] = jnp.maximum(o, 0.0)

    nspec = pl.BlockSpec((Nb, 128), lambda i: (i, 0))
    dspec = pl.BlockSpec((Nb, 16), lambda i: (i, 0))
    n2spec = pl.BlockSpec((Nb, 128), lambda i: (i + N // Nb, 0))
    d2spec = pl.BlockSpec((Nb, 16), lambda i: (i + N // Nb, 0))
    return pl.pallas_call(
        body,
        grid=(N // Nb,),
        in_specs=[nspec, n2spec, dspec, d2spec,
                  pl.BlockSpec((16, 128), lambda i: (0, 0)),
                  pl.BlockSpec((1, 128), lambda i: (0, 0)),
                  pl.BlockSpec((128, 128), lambda i: (0, 0)),
                  pl.BlockSpec((1, 128), lambda i: (0, 0))],
        out_specs=nspec,
        out_shape=jax.ShapeDtypeStruct((N, 128), jnp.float32),
    )(num, num, den, den, expand, bias, wl, bl)


def _make_edge_pass(n_heads):
    head_of = [j if n_heads == 8 else 0 for j in range(8)]
    mesh = plsc.VectorSubcoreMesh(core_axis_name="c", subcore_axis_name="s")

    @functools.partial(
        pl.kernel,
        mesh=mesh,
        compiler_params=pltpu.CompilerParams(use_tc_tiling_on_sc=False),
        out_type=[jax.ShapeDtypeStruct((NC * N, 128), jnp.float32),
                  jax.ShapeDtypeStruct((NC * N, 16), jnp.float32)],
        scratch_types=[
            pltpu.VMEM_SHARED((N, 128), jnp.float32),
            pltpu.VMEM_SHARED((N, 16), jnp.float32),
            pltpu.VMEM((CH,), jnp.int32),
            pltpu.VMEM((CH,), jnp.int32),
            pltpu.VMEM((CH,), jnp.int32),
            pltpu.VMEM((CH,), jnp.int32),
            pltpu.VMEM((CH,), jnp.int32),
            pltpu.VMEM((CH,), jnp.int32),
            pltpu.VMEM((CH,), jnp.int32),
            pltpu.VMEM((CH,), jnp.int32),
            pltpu.VMEM((CH, 128), jnp.float32),
            pltpu.VMEM((CH, 128), jnp.float32),
            pltpu.VMEM((CH, 16), jnp.float32),
            pltpu.VMEM((CH, 16), jnp.float32),
            pltpu.VMEM((CH, 16), jnp.float32),
            pltpu.VMEM((CH, 16), jnp.float32),
            pltpu.VMEM((CH, 16), jnp.float32),
            pltpu.VMEM((CH, 16), jnp.float32),
            pltpu.VMEM((CH, 16), jnp.float32),
            pltpu.VMEM((CH, 16), jnp.float32),
        ] + [pltpu.SemaphoreType.DMA] * 6,
    )
    def edge_pass(src_hbm, dst_hbm, h_hbm, als_hbm, ald_hbm, ale_hbm,
                  num_out, den_out,
                  num_sh, den_sh,
                  srcI0, srcI1, srcI2, srcI3, dstI0, dstI1, dstI2, dstI3,
                  hg0, hg1, ale0, ale1, als0, als1, ald0, ald1, wb0, wb1,
                  bsem0, bsem1, gsem0, gsem1, ssem0, ssem1):
        c = lax.axis_index("c")
        s = lax.axis_index("s")
        wid = c * NS + s
        srcI = [srcI0, srcI1, srcI2, srcI3]
        dstI = [dstI0, dstI1, dstI2, dstI3]
        hg = [hg0, hg1]
        alel = [ale0, ale1]
        alsg = [als0, als1]
        aldg = [ald0, ald1]
        wb = [wb0, wb1]
        bsem = [bsem0, bsem1]
        gsem = [gsem0, gsem1]
        ssem = [ssem0, ssem1]

        def base_copies(g, r, p):
            row = wid * CPT + g
            return [
                pltpu.make_async_copy(src_hbm.at[row], srcI[r], bsem[p]),
                pltpu.make_async_copy(dst_hbm.at[row], dstI[r], bsem[p]),
                pltpu.make_async_copy(ale_hbm.at[row], alel[p], bsem[p]),
            ]

        def gather_copies(p, r):
            return [
                pltpu.make_async_copy(als_hbm.at[srcI[r]], alsg[p], gsem[p]),
                pltpu.make_async_copy(ald_hbm.at[dstI[r]], aldg[p], gsem[p]),
                pltpu.make_async_copy(h_hbm.at[srcI[r]], hg[p], gsem[p]),
            ]

        def scatter_copies(p, r):
            return [
                pltpu.make_async_copy(hg[p], num_sh.at[dstI[r]], ssem[p]),
                pltpu.make_async_copy(wb[p], den_sh.at[dstI[r]], ssem[p]),
            ]

        def fire_scatters(p, r):
            pltpu.async_copy(hg[p], num_sh.at[dstI[r]], ssem[p], add=True)
            pltpu.async_copy(wb[p], den_sh.at[dstI[r]], ssem[p], add=True)

        zv = jnp.zeros((16,), jnp.float32)

        def zrow(r, _):
            for j in range(8):
                hg0[r, pl.ds(j * 16, 16)] = zv
            wb0[r, :] = zv
            return 0

        lax.fori_loop(0, CH, zrow, 0)
        r0 = s * NPT
        done = 0
        while done < NPT:
            step = min(CH, NPT - done)
            pltpu.sync_copy(hg0.at[pl.ds(0, step)], num_sh.at[pl.ds(r0 + done, step)])
            pltpu.sync_copy(wb0.at[pl.ds(0, step)], den_sh.at[pl.ds(r0 + done, step)])
            done += step

        @pl.when(s == 0)
        def _zero_tail():
            pltpu.sync_copy(hg0.at[pl.ds(0, NTAIL)], num_sh.at[pl.ds(NS * NPT, NTAIL)])
            pltpu.sync_copy(wb0.at[pl.ds(0, NTAIL)], den_sh.at[pl.ds(NS * NPT, NTAIL)])

        plsc.subcore_barrier()

        for cp in base_copies(0, 0, 0) + base_copies(1, 1, 1):
            cp.start()
        for cp in base_copies(0, 0, 0):
            cp.wait()
        for cp in gather_copies(0, 0):
            cp.start()

        def half(g, b):
            p = b % 2
            q = 1 - p
            rn = (b + 1) % 4
            rp = (b - 1) % 4
            rf = (b + 2) % 4

            @pl.when(g + 1 < CPT)
            def _advance_other():
                for cp in base_copies(g + 1, rn, q):
                    cp.wait()

                @pl.when(g >= 1)
                def _drain_scatter_q():
                    for cp in scatter_copies(q, rp):
                        cp.wait()

                for cp in gather_copies(q, rn):
                    cp.start()

            for cp in gather_copies(p, b):
                cp.wait()

            def wpass(e, _):
                a = alsg[p][e, :] + aldg[p][e, :] + alel[p][e, :]
                a = jnp.where(a >= 0.0, a, 0.2 * a)
                wb[p][e, :] = jnp.exp(a)
                return 0

            def mpass(e, _):
                w = wb[p][e, :]
                for j in range(8):
                    hg[p][e, pl.ds(j * 16, 16)] = (
                        hg[p][e, pl.ds(j * 16, 16)] * w[head_of[j]])
                return 0

            lax.fori_loop(0, CH, wpass, 0)
            lax.fori_loop(0, CH, mpass, 0, unroll=4)
            fire_scatters(p, b)

            @pl.when(g + 2 < CPT)
            def _prefetch_base():
                for cp in base_copies(g + 2, rf, p):
                    cp.start()

        def pipe(i, _):
            for b in range(4):
                half(4 * i + b, b)
            return 0

        lax.fori_loop(0, CPT // 4, pipe, 0)
        for cp in scatter_copies(0, 2) + scatter_copies(1, 3):
            cp.wait()
        plsc.subcore_barrier()

        pltpu.sync_copy(num_sh.at[pl.ds(r0, NPT)], num_out.at[pl.ds(c * N + r0, NPT)])
        pltpu.sync_copy(den_sh.at[pl.ds(r0, NPT)], den_out.at[pl.ds(c * N + r0, NPT)])

        @pl.when(s == 0)
        def _write_tail():
            pltpu.sync_copy(num_sh.at[pl.ds(NS * NPT, NTAIL)],
                            num_out.at[pl.ds(c * N + NS * NPT, NTAIL)])
            pltpu.sync_copy(den_sh.at[pl.ds(NS * NPT, NTAIL)],
                            den_out.at[pl.ds(c * N + NS * NPT, NTAIL)])

    return edge_pass


_edge_pass_h8 = _make_edge_pass(8)
_edge_pass_h1 = _make_edge_pass(1)


def _fold_we(We, a_e, H, C):
    wf = (We.reshape(We.shape[0], H, C) * a_e[None]).sum(-1)
    return jnp.pad(wf, ((0, 0), (0, 16 - H)))

def _a_mat(a, H, C):
    m = jnp.zeros((H * C, 16), jnp.float32)
    for h in range(H):
        m = m.at[h * C:(h + 1) * C, h].set(a[h])
    return m

def _expand_mat(H):
    m = np.zeros((16, 128), np.float32)
    C = 128 // H
    for h in range(H):
        m[h, h * C:(h + 1) * C] = 1.0
    return jnp.asarray(m)


def kernel(x, edge_index, edge_attr, W1, We1, as1, ad1, ae1, b1,
           W2, We2, as2, ad2, ae2, b2, W3, We3, as3, ad3, ae3, b3, Wl, bl):
    src = edge_index[0].reshape(E // CH, CH)
    dst = edge_index[1].reshape(E // CH, CH)

    eap = edge_attr.reshape(E // 8, 128)
    eye8 = jnp.eye(8, dtype=jnp.float32)
    ale1, ale2, ale3 = _edge_logits(
        eap, jnp.kron(eye8, _fold_we(We1, ae1, 8, 16)),
        jnp.kron(eye8, _fold_we(We2, ae2, 8, 16)),
        jnp.kron(eye8, _fold_we(We3, ae3, 1, 128)))
    ale1 = ale1.reshape(E // CH, CH, 16)
    ale2 = ale2.reshape(E // CH, CH, 16)
    ale3 = ale3.reshape(E // CH, CH, 16)

    ex8 = _expand_mat(8)
    ex1 = _expand_mat(1)

    h, als, ald = _node_dense(x, W1, _a_mat(as1, 8, 16), _a_mat(ad1, 8, 16))
    num, den = _edge_pass_h8(src, dst, h, als, ald, ale1)

    h, als, ald = _node_dense(
        None, W2, _a_mat(as2, 8, 16), _a_mat(ad2, 8, 16),
        prev=(num, den, ex8, b1.reshape(1, 128)))
    num, den = _edge_pass_h8(src, dst, h, als, ald, ale2)

    h, als, ald = _node_dense(
        None, W3, _a_mat(as3, 1, 128), _a_mat(ad3, 1, 128),
        prev=(num, den, ex8, b2.reshape(1, 128)))
    num, den = _edge_pass_h1(src, dst, h, als, ald, ale3)

    return _final_dense(num, den, ex1,
                        b3.reshape(1, 128), Wl, bl.reshape(1, 128))


# speedup vs baseline: 1.3281x; 1.0103x over previous
import functools

import jax
import jax.numpy as jnp
import numpy as np
from jax import lax
from jax.experimental import pallas as pl
from jax.experimental.pallas import tpu as pltpu
from jax.experimental.pallas import tpu_sc as plsc

N = 10000
E = 320000
D = 128

NC = 2
NS = 16
NW = NC * NS
EW = E // NW
CH = 100
NPT = 624
NTAIL = N - NS * NPT
CPT = EW // CH


def _edge_logits(eap, wf1, wf2, wf3):
    Eb = 2000

    def body(ea_ref, w1_ref, w2_ref, w3_ref, o1_ref, o2_ref, o3_ref):
        a = ea_ref[...]
        o1_ref[...] = jnp.dot(a, w1_ref[...], preferred_element_type=jnp.float32)
        o2_ref[...] = jnp.dot(a, w2_ref[...], preferred_element_type=jnp.float32)
        o3_ref[...] = jnp.dot(a, w3_ref[...], preferred_element_type=jnp.float32)

    wspec = pl.BlockSpec((128, 128), lambda i: (0, 0))
    espec = pl.BlockSpec((Eb, 128), lambda i: (i, 0))
    out = pl.pallas_call(
        body,
        grid=(E // 8 // Eb,),
        in_specs=[espec, wspec, wspec, wspec],
        out_specs=[espec] * 3,
        out_shape=[jax.ShapeDtypeStruct((E // 8, 128), jnp.float32)] * 3,
    )(eap, wf1, wf2, wf3)
    return out


def _node_dense(x, w, ams, amd, prev=None):
    Nb = 1000
    wspec = pl.BlockSpec((128, 128), lambda i: (0, 0))
    aspec = pl.BlockSpec((128, 16), lambda i: (0, 0))
    nspec = pl.BlockSpec((Nb, 128), lambda i: (i, 0))
    dspec = pl.BlockSpec((Nb, 16), lambda i: (i, 0))
    n2spec = pl.BlockSpec((Nb, 128), lambda i: (i + N // Nb, 0))
    d2spec = pl.BlockSpec((Nb, 16), lambda i: (i + N // Nb, 0))

    if prev is None:
        def body(x_ref, w_ref, ams_ref, amd_ref, h_ref, als_ref, ald_ref):
            h = jnp.dot(x_ref[...], w_ref[...], preferred_element_type=jnp.float32)
            h_ref[...] = h
            als_ref[...] = jnp.dot(h, ams_ref[...], preferred_element_type=jnp.float32)
            ald_ref[...] = jnp.dot(h, amd_ref[...], preferred_element_type=jnp.float32)

        in_specs = [nspec, wspec, aspec, aspec]
        args = (x, w, ams, amd)
    else:
        num, den, expand, bias = prev

        def body(n0_ref, n1_ref, d0_ref, d1_ref, ex_ref, b_ref, w_ref,
                 ams_ref, amd_ref, h_ref, als_ref, ald_ref):
            den = d0_ref[...] + d1_ref[...]
            de = jnp.dot(den, ex_ref[...], preferred_element_type=jnp.float32)
            xv = (n0_ref[...] + n1_ref[...]) / (de + 1e-16) + b_ref[...]
            xv = jnp.maximum(xv, 0.0)
            h = jnp.dot(xv, w_ref[...], preferred_element_type=jnp.float32)
            h_ref[...] = h
            als_ref[...] = jnp.dot(h, ams_ref[...], preferred_element_type=jnp.float32)
            ald_ref[...] = jnp.dot(h, amd_ref[...], preferred_element_type=jnp.float32)

        in_specs = [nspec, n2spec, dspec, d2spec,
                    pl.BlockSpec((16, 128), lambda i: (0, 0)),
                    pl.BlockSpec((1, 128), lambda i: (0, 0)),
                    wspec, aspec, aspec]
        args = (num, num, den, den, expand, bias, w, ams, amd)

    return pl.pallas_call(
        body,
        grid=(N // Nb,),
        in_specs=in_specs,
        out_specs=[nspec, dspec, dspec],
        out_shape=[jax.ShapeDtypeStruct((N, 128), jnp.float32),
                   jax.ShapeDtypeStruct((N, 16), jnp.float32),
                   jax.ShapeDtypeStruct((N, 16), jnp.float32)],
    )(*args)


def _final_dense(num, den, expand, bias, wl, bl):
    Nb = 1000

    def body(n0_ref, n1_ref, d0_ref, d1_ref, ex_ref, b_ref, w_ref, bl_ref, o_ref):
        den = d0_ref[...] + d1_ref[...]
        de = jnp.dot(den, ex_ref[...], preferred_element_type=jnp.float32)
        xv = (n0_ref[...] + n1_ref[...]) / (de + 1e-16) + b_ref[...]
        xv = jnp.maximum(xv, 0.0)
        o = jnp.dot(xv, w_ref[...], preferred_element_type=jnp.float32) + bl_ref[...]
        o_ref[...] = jnp.maximum(o, 0.0)

    nspec = pl.BlockSpec((Nb, 128), lambda i: (i, 0))
    dspec = pl.BlockSpec((Nb, 16), lambda i: (i, 0))
    n2spec = pl.BlockSpec((Nb, 128), lambda i: (i + N // Nb, 0))
    d2spec = pl.BlockSpec((Nb, 16), lambda i: (i + N // Nb, 0))
    return pl.pallas_call(
        body,
        grid=(N // Nb,),
        in_specs=[nspec, n2spec, dspec, d2spec,
                  pl.BlockSpec((16, 128), lambda i: (0, 0)),
                  pl.BlockSpec((1, 128), lambda i: (0, 0)),
                  pl.BlockSpec((128, 128), lambda i: (0, 0)),
                  pl.BlockSpec((1, 128), lambda i: (0, 0))],
        out_specs=nspec,
        out_shape=jax.ShapeDtypeStruct((N, 128), jnp.float32),
    )(num, num, den, den, expand, bias, wl, bl)


def _make_edge_pass(n_heads):
    head_of = [j if n_heads == 8 else 0 for j in range(8)]
    mesh = plsc.VectorSubcoreMesh(core_axis_name="c", subcore_axis_name="s")

    @functools.partial(
        pl.kernel,
        mesh=mesh,
        compiler_params=pltpu.CompilerParams(use_tc_tiling_on_sc=False),
        out_type=[jax.ShapeDtypeStruct((NC * N, 128), jnp.float32),
                  jax.ShapeDtypeStruct((NC * N, 16), jnp.float32)],
        scratch_types=[
            pltpu.VMEM_SHARED((N, 128), jnp.float32),
            pltpu.VMEM_SHARED((N, 16), jnp.float32),
            pltpu.VMEM((CH,), jnp.int32),
            pltpu.VMEM((CH,), jnp.int32),
            pltpu.VMEM((CH,), jnp.int32),
            pltpu.VMEM((CH,), jnp.int32),
            pltpu.VMEM((CH,), jnp.int32),
            pltpu.VMEM((CH,), jnp.int32),
            pltpu.VMEM((CH,), jnp.int32),
            pltpu.VMEM((CH,), jnp.int32),
            pltpu.VMEM((CH, 128), jnp.float32),
            pltpu.VMEM((CH, 128), jnp.float32),
            pltpu.VMEM((CH, 16), jnp.float32),
            pltpu.VMEM((CH, 16), jnp.float32),
            pltpu.VMEM((CH, 16), jnp.float32),
            pltpu.VMEM((CH, 16), jnp.float32),
            pltpu.VMEM((CH, 16), jnp.float32),
            pltpu.VMEM((CH, 16), jnp.float32),
            pltpu.VMEM((CH, 16), jnp.float32),
            pltpu.VMEM((CH, 16), jnp.float32),
        ] + [pltpu.SemaphoreType.DMA] * 6,
    )
    def edge_pass(src_hbm, dst_hbm, h_hbm, als_hbm, ald_hbm, ale_hbm,
                  num_out, den_out,
                  num_sh, den_sh,
                  srcI0, srcI1, srcI2, srcI3, dstI0, dstI1, dstI2, dstI3,
                  hg0, hg1, ale0, ale1, als0, als1, ald0, ald1, wb0, wb1,
                  bsem0, bsem1, gsem0, gsem1, ssem0, ssem1):
        c = lax.axis_index("c")
        s = lax.axis_index("s")
        wid = c * NS + s
        srcI = [srcI0, srcI1, srcI2, srcI3]
        dstI = [dstI0, dstI1, dstI2, dstI3]
        hg = [hg0, hg1]
        alel = [ale0, ale1]
        alsg = [als0, als1]
        aldg = [ald0, ald1]
        wb = [wb0, wb1]
        bsem = [bsem0, bsem1]
        gsem = [gsem0, gsem1]
        ssem = [ssem0, ssem1]

        def base_copies(g, r, p):
            row = wid * CPT + g
            return [
                pltpu.make_async_copy(src_hbm.at[row], srcI[r], bsem[p]),
                pltpu.make_async_copy(dst_hbm.at[row], dstI[r], bsem[p]),
                pltpu.make_async_copy(ale_hbm.at[row], alel[p], bsem[p]),
            ]

        def gather_copies(p, r):
            return [
                pltpu.make_async_copy(als_hbm.at[srcI[r]], alsg[p], gsem[p]),
                pltpu.make_async_copy(ald_hbm.at[dstI[r]], aldg[p], gsem[p]),
                pltpu.make_async_copy(h_hbm.at[srcI[r]], hg[p], gsem[p]),
            ]

        def scatter_copies(p, r):
            return [
                pltpu.make_async_copy(hg[p], num_sh.at[dstI[r]], ssem[p]),
                pltpu.make_async_copy(wb[p], den_sh.at[dstI[r]], ssem[p]),
            ]

        def fire_scatters(p, r):
            pltpu.async_copy(hg[p], num_sh.at[dstI[r]], ssem[p], add=True)
            pltpu.async_copy(wb[p], den_sh.at[dstI[r]], ssem[p], add=True)

        zv = jnp.zeros((16,), jnp.float32)

        def zrow(r, _):
            for j in range(8):
                hg0[r, pl.ds(j * 16, 16)] = zv
            wb0[r, :] = zv
            return 0

        lax.fori_loop(0, CH, zrow, 0)
        r0 = s * NPT
        done = 0
        while done < NPT:
            step = min(CH, NPT - done)
            pltpu.sync_copy(hg0.at[pl.ds(0, step)], num_sh.at[pl.ds(r0 + done, step)])
            pltpu.sync_copy(wb0.at[pl.ds(0, step)], den_sh.at[pl.ds(r0 + done, step)])
            done += step

        @pl.when(s == 0)
        def _zero_tail():
            pltpu.sync_copy(hg0.at[pl.ds(0, NTAIL)], num_sh.at[pl.ds(NS * NPT, NTAIL)])
            pltpu.sync_copy(wb0.at[pl.ds(0, NTAIL)], den_sh.at[pl.ds(NS * NPT, NTAIL)])

        plsc.subcore_barrier()

        for cp in base_copies(0, 0, 0) + base_copies(1, 1, 1):
            cp.start()
        for cp in base_copies(0, 0, 0):
            cp.wait()
        for cp in gather_copies(0, 0):
            cp.start()

        def half(g, b):
            p = b % 2
            q = 1 - p
            rn = (b + 1) % 4
            rp = (b - 1) % 4
            rf = (b + 2) % 4

            @pl.when(g + 1 < CPT)
            def _advance_other():
                for cp in base_copies(g + 1, rn, q):
                    cp.wait()

                @pl.when(g >= 1)
                def _drain_scatter_q():
                    for cp in scatter_copies(q, rp):
                        cp.wait()

                for cp in gather_copies(q, rn):
                    cp.start()

            for cp in gather_copies(p, b):
                cp.wait()

            def wpass(e, _):
                a = alsg[p][e, :] + aldg[p][e, :] + alel[p][e, :]
                a = jnp.where(a >= 0.0, a, 0.2 * a)
                wb[p][e, :] = jnp.exp(a)
                return 0

            def mpass(e, _):
                w = wb[p][e, :]
                for j in range(8):
                    hg[p][e, pl.ds(j * 16, 16)] = (
                        hg[p][e, pl.ds(j * 16, 16)] * w[head_of[j]])
                return 0

            lax.fori_loop(0, CH, wpass, 0)
            lax.fori_loop(0, CH, mpass, 0, unroll=2)
            fire_scatters(p, b)

            @pl.when(g + 2 < CPT)
            def _prefetch_base():
                for cp in base_copies(g + 2, rf, p):
                    cp.start()

        def pipe(i, _):
            for b in range(4):
                half(4 * i + b, b)
            return 0

        lax.fori_loop(0, CPT // 4, pipe, 0)
        for cp in scatter_copies(0, 2) + scatter_copies(1, 3):
            cp.wait()
        plsc.subcore_barrier()

        pltpu.sync_copy(num_sh.at[pl.ds(r0, NPT)], num_out.at[pl.ds(c * N + r0, NPT)])
        pltpu.sync_copy(den_sh.at[pl.ds(r0, NPT)], den_out.at[pl.ds(c * N + r0, NPT)])

        @pl.when(s == 0)
        def _write_tail():
            pltpu.sync_copy(num_sh.at[pl.ds(NS * NPT, NTAIL)],
                            num_out.at[pl.ds(c * N + NS * NPT, NTAIL)])
            pltpu.sync_copy(den_sh.at[pl.ds(NS * NPT, NTAIL)],
                            den_out.at[pl.ds(c * N + NS * NPT, NTAIL)])

    return edge_pass


_edge_pass_h8 = _make_edge_pass(8)
_edge_pass_h1 = _make_edge_pass(1)


def _fold_we(We, a_e, H, C):
    wf = (We.reshape(We.shape[0], H, C) * a_e[None]).sum(-1)
    return jnp.pad(wf, ((0, 0), (0, 16 - H)))

def _a_mat(a, H, C):
    m = jnp.zeros((H * C, 16), jnp.float32)
    for h in range(H):
        m = m.at[h * C:(h + 1) * C, h].set(a[h])
    return m

def _expand_mat(H):
    m = np.zeros((16, 128), np.float32)
    C = 128 // H
    for h in range(H):
        m[h, h * C:(h + 1) * C] = 1.0
    return jnp.asarray(m)


def kernel(x, edge_index, edge_attr, W1, We1, as1, ad1, ae1, b1,
           W2, We2, as2, ad2, ae2, b2, W3, We3, as3, ad3, ae3, b3, Wl, bl):
    src = edge_index[0].reshape(E // CH, CH)
    dst = edge_index[1].reshape(E // CH, CH)

    eap = edge_attr.reshape(E // 8, 128)
    eye8 = jnp.eye(8, dtype=jnp.float32)
    ale1, ale2, ale3 = _edge_logits(
        eap, jnp.kron(eye8, _fold_we(We1, ae1, 8, 16)),
        jnp.kron(eye8, _fold_we(We2, ae2, 8, 16)),
        jnp.kron(eye8, _fold_we(We3, ae3, 1, 128)))
    ale1 = ale1.reshape(E // CH, CH, 16)
    ale2 = ale2.reshape(E // CH, CH, 16)
    ale3 = ale3.reshape(E // CH, CH, 16)

    ex8 = _expand_mat(8)
    ex1 = _expand_mat(1)

    h, als, ald = _node_dense(x, W1, _a_mat(as1, 8, 16), _a_mat(ad1, 8, 16))
    num, den = _edge_pass_h8(src, dst, h, als, ald, ale1)

    h, als, ald = _node_dense(
        None, W2, _a_mat(as2, 8, 16), _a_mat(ad2, 8, 16),
        prev=(num, den, ex8, b1.reshape(1, 128)))
    num, den = _edge_pass_h8(src, dst, h, als, ald, ale2)

    h, als, ald = _node_dense(
        None, W3, _a_mat(as3, 1, 128), _a_mat(ad3, 1, 128),
        prev=(num, den, ex8, b2.reshape(1, 128)))
    num, den = _edge_pass_h1(src, dst, h, als, ald, ale3)

    return _final_dense(num, den, ex1,
                        b3.reshape(1, 128), Wl, bl.reshape(1, 128))


# speedup vs baseline: 1.3316x; 1.0026x over previous
import functools

import jax
import jax.numpy as jnp
import numpy as np
from jax import lax
from jax.experimental import pallas as pl
from jax.experimental.pallas import tpu as pltpu
from jax.experimental.pallas import tpu_sc as plsc

N = 10000
E = 320000
D = 128

NC = 2
NS = 16
NW = NC * NS
EW = E // NW
CH = 100
NPT = 624
NTAIL = N - NS * NPT
CPT = EW // CH


def _edge_logits(eap, wf1, wf2, wf3):
    Eb = 2000

    def body(ea_ref, w1_ref, w2_ref, w3_ref, o1_ref, o2_ref, o3_ref):
        a = ea_ref[...]
        o1_ref[...] = jnp.dot(a, w1_ref[...], preferred_element_type=jnp.float32)
        o2_ref[...] = jnp.dot(a, w2_ref[...], preferred_element_type=jnp.float32)
        o3_ref[...] = jnp.dot(a, w3_ref[...], preferred_element_type=jnp.float32)

    wspec = pl.BlockSpec((128, 128), lambda i: (0, 0))
    espec = pl.BlockSpec((Eb, 128), lambda i: (i, 0))

    def body1(ea_ref, w1_ref, o1_ref):
        o1_ref[...] = jnp.dot(ea_ref[...], w1_ref[...],
                              preferred_element_type=jnp.float32)

    def body23(ea_ref, w2_ref, w3_ref, o2_ref, o3_ref):
        a = ea_ref[...]
        o2_ref[...] = jnp.dot(a, w2_ref[...], preferred_element_type=jnp.float32)
        o3_ref[...] = jnp.dot(a, w3_ref[...], preferred_element_type=jnp.float32)

    o1 = pl.pallas_call(
        body1,
        grid=(E // 8 // Eb,),
        in_specs=[espec, wspec],
        out_specs=espec,
        out_shape=jax.ShapeDtypeStruct((E // 8, 128), jnp.float32),
    )(eap, wf1)
    o2, o3 = pl.pallas_call(
        body23,
        grid=(E // 8 // Eb,),
        in_specs=[espec, wspec, wspec],
        out_specs=[espec] * 2,
        out_shape=[jax.ShapeDtypeStruct((E // 8, 128), jnp.float32)] * 2,
    )(eap, wf2, wf3)
    return o1, o2, o3


def _node_dense(x, w, ams, amd, prev=None):
    Nb = 1000
    wspec = pl.BlockSpec((128, 128), lambda i: (0, 0))
    aspec = pl.BlockSpec((128, 16), lambda i: (0, 0))
    nspec = pl.BlockSpec((Nb, 128), lambda i: (i, 0))
    dspec = pl.BlockSpec((Nb, 16), lambda i: (i, 0))
    n2spec = pl.BlockSpec((Nb, 128), lambda i: (i + N // Nb, 0))
    d2spec = pl.BlockSpec((Nb, 16), lambda i: (i + N // Nb, 0))

    if prev is None:
        def body(x_ref, w_ref, ams_ref, amd_ref, h_ref, als_ref, ald_ref):
            h = jnp.dot(x_ref[...], w_ref[...], preferred_element_type=jnp.float32)
            h_ref[...] = h
            als_ref[...] = jnp.dot(h, ams_ref[...], preferred_element_type=jnp.float32)
            ald_ref[...] = jnp.dot(h, amd_ref[...], preferred_element_type=jnp.float32)

        in_specs = [nspec, wspec, aspec, aspec]
        args = (x, w, ams, amd)
    else:
        num, den, expand, bias = prev

        def body(n0_ref, n1_ref, d0_ref, d1_ref, ex_ref, b_ref, w_ref,
                 ams_ref, amd_ref, h_ref, als_ref, ald_ref):
            den = d0_ref[...] + d1_ref[...]
            de = jnp.dot(den, ex_ref[...], preferred_element_type=jnp.float32)
            xv = (n0_ref[...] + n1_ref[...]) / (de + 1e-16) + b_ref[...]
            xv = jnp.maximum(xv, 0.0)
            h = jnp.dot(xv, w_ref[...], preferred_element_type=jnp.float32)
            h_ref[...] = h
            als_ref[...] = jnp.dot(h, ams_ref[...], preferred_element_type=jnp.float32)
            ald_ref[...] = jnp.dot(h, amd_ref[...], preferred_element_type=jnp.float32)

        in_specs = [nspec, n2spec, dspec, d2spec,
                    pl.BlockSpec((16, 128), lambda i: (0, 0)),
                    pl.BlockSpec((1, 128), lambda i: (0, 0)),
                    wspec, aspec, aspec]
        args = (num, num, den, den, expand, bias, w, ams, amd)

    return pl.pallas_call(
        body,
        grid=(N // Nb,),
        in_specs=in_specs,
        out_specs=[nspec, dspec, dspec],
        out_shape=[jax.ShapeDtypeStruct((N, 128), jnp.float32),
                   jax.ShapeDtypeStruct((N, 16), jnp.float32),
                   jax.ShapeDtypeStruct((N, 16), jnp.float32)],
    )(*args)


def _final_dense(num, den, expand, bias, wl, bl):
    Nb = 1000

    def body(n0_ref, n1_ref, d0_ref, d1_ref, ex_ref, b_ref, w_ref, bl_ref, o_ref):
        den = d0_ref[...] + d1_ref[...]
        de = jnp.dot(den, ex_ref[...], preferred_element_type=jnp.float32)
        xv = (n0_ref[...] + n1_ref[...]) / (de + 1e-16) + b_ref[...]
        xv = jnp.maximum(xv, 0.0)
        o = jnp.dot(xv, w_ref[...], preferred_element_type=jnp.float32) + bl_ref[...]
        o_ref[...] = jnp.maximum(o, 0.0)

    nspec = pl.BlockSpec((Nb, 128), lambda i: (i, 0))
    dspec = pl.BlockSpec((Nb, 16), lambda i: (i, 0))
    n2spec = pl.BlockSpec((Nb, 128), lambda i: (i + N // Nb, 0))
    d2spec = pl.BlockSpec((Nb, 16), lambda i: (i + N // Nb, 0))
    return pl.pallas_call(
        body,
        grid=(N // Nb,),
        in_specs=[nspec, n2spec, dspec, d2spec,
                  pl.BlockSpec((16, 128), lambda i: (0, 0)),
                  pl.BlockSpec((1, 128), lambda i: (0, 0)),
                  pl.BlockSpec((128, 128), lambda i: (0, 0)),
                  pl.BlockSpec((1, 128), lambda i: (0, 0))],
        out_specs=nspec,
        out_shape=jax.ShapeDtypeStruct((N, 128), jnp.float32),
    )(num, num, den, den, expand, bias, wl, bl)


def _make_edge_pass(n_heads):
    head_of = [j if n_heads == 8 else 0 for j in range(8)]
    mesh = plsc.VectorSubcoreMesh(core_axis_name="c", subcore_axis_name="s")

    @functools.partial(
        pl.kernel,
        mesh=mesh,
        compiler_params=pltpu.CompilerParams(use_tc_tiling_on_sc=False),
        out_type=[jax.ShapeDtypeStruct((NC * N, 128), jnp.float32),
                  jax.ShapeDtypeStruct((NC * N, 16), jnp.float32)],
        scratch_types=[
            pltpu.VMEM_SHARED((N, 128), jnp.float32),
            pltpu.VMEM_SHARED((N, 16), jnp.float32),
            pltpu.VMEM((CH,), jnp.int32),
            pltpu.VMEM((CH,), jnp.int32),
            pltpu.VMEM((CH,), jnp.int32),
            pltpu.VMEM((CH,), jnp.int32),
            pltpu.VMEM((CH,), jnp.int32),
            pltpu.VMEM((CH,), jnp.int32),
            pltpu.VMEM((CH,), jnp.int32),
            pltpu.VMEM((CH,), jnp.int32),
            pltpu.VMEM((CH, 128), jnp.float32),
            pltpu.VMEM((CH, 128), jnp.float32),
            pltpu.VMEM((CH, 16), jnp.float32),
            pltpu.VMEM((CH, 16), jnp.float32),
            pltpu.VMEM((CH, 16), jnp.float32),
            pltpu.VMEM((CH, 16), jnp.float32),
            pltpu.VMEM((CH, 16), jnp.float32),
            pltpu.VMEM((CH, 16), jnp.float32),
            pltpu.VMEM((CH, 16), jnp.float32),
            pltpu.VMEM((CH, 16), jnp.float32),
        ] + [pltpu.SemaphoreType.DMA] * 6,
    )
    def edge_pass(src_hbm, dst_hbm, h_hbm, als_hbm, ald_hbm, ale_hbm,
                  num_out, den_out,
                  num_sh, den_sh,
                  srcI0, srcI1, srcI2, srcI3, dstI0, dstI1, dstI2, dstI3,
                  hg0, hg1, ale0, ale1, als0, als1, ald0, ald1, wb0, wb1,
                  bsem0, bsem1, gsem0, gsem1, ssem0, ssem1):
        c = lax.axis_index("c")
        s = lax.axis_index("s")
        wid = c * NS + s
        srcI = [srcI0, srcI1, srcI2, srcI3]
        dstI = [dstI0, dstI1, dstI2, dstI3]
        hg = [hg0, hg1]
        alel = [ale0, ale1]
        alsg = [als0, als1]
        aldg = [ald0, ald1]
        wb = [wb0, wb1]
        bsem = [bsem0, bsem1]
        gsem = [gsem0, gsem1]
        ssem = [ssem0, ssem1]

        def base_copies(g, r, p):
            row = wid * CPT + g
            return [
                pltpu.make_async_copy(src_hbm.at[row], srcI[r], bsem[p]),
                pltpu.make_async_copy(dst_hbm.at[row], dstI[r], bsem[p]),
                pltpu.make_async_copy(ale_hbm.at[row], alel[p], bsem[p]),
            ]

        def gather_copies(p, r):
            return [
                pltpu.make_async_copy(als_hbm.at[srcI[r]], alsg[p], gsem[p]),
                pltpu.make_async_copy(ald_hbm.at[dstI[r]], aldg[p], gsem[p]),
                pltpu.make_async_copy(h_hbm.at[srcI[r]], hg[p], gsem[p]),
            ]

        def scatter_copies(p, r):
            return [
                pltpu.make_async_copy(hg[p], num_sh.at[dstI[r]], ssem[p]),
                pltpu.make_async_copy(wb[p], den_sh.at[dstI[r]], ssem[p]),
            ]

        def fire_scatters(p, r):
            pltpu.async_copy(hg[p], num_sh.at[dstI[r]], ssem[p], add=True)
            pltpu.async_copy(wb[p], den_sh.at[dstI[r]], ssem[p], add=True)

        zv = jnp.zeros((16,), jnp.float32)

        def zrow(r, _):
            for j in range(8):
                hg0[r, pl.ds(j * 16, 16)] = zv
            wb0[r, :] = zv
            return 0

        lax.fori_loop(0, CH, zrow, 0)
        r0 = s * NPT
        done = 0
        while done < NPT:
            step = min(CH, NPT - done)
            pltpu.sync_copy(hg0.at[pl.ds(0, step)], num_sh.at[pl.ds(r0 + done, step)])
            pltpu.sync_copy(wb0.at[pl.ds(0, step)], den_sh.at[pl.ds(r0 + done, step)])
            done += step

        @pl.when(s == 0)
        def _zero_tail():
            pltpu.sync_copy(hg0.at[pl.ds(0, NTAIL)], num_sh.at[pl.ds(NS * NPT, NTAIL)])
            pltpu.sync_copy(wb0.at[pl.ds(0, NTAIL)], den_sh.at[pl.ds(NS * NPT, NTAIL)])

        plsc.subcore_barrier()

        for cp in base_copies(0, 0, 0) + base_copies(1, 1, 1):
            cp.start()
        for cp in base_copies(0, 0, 0):
            cp.wait()
        for cp in gather_copies(0, 0):
            cp.start()

        def half(g, b):
            p = b % 2
            q = 1 - p
            rn = (b + 1) % 4
            rp = (b - 1) % 4
            rf = (b + 2) % 4

            @pl.when(g + 1 < CPT)
            def _advance_other():
                for cp in base_copies(g + 1, rn, q):
                    cp.wait()

                @pl.when(g >= 1)
                def _drain_scatter_q():
                    for cp in scatter_copies(q, rp):
                        cp.wait()

                for cp in gather_copies(q, rn):
                    cp.start()

            for cp in gather_copies(p, b):
                cp.wait()

            def wpass(e, _):
                a = alsg[p][e, :] + aldg[p][e, :] + alel[p][e, :]
                a = jnp.where(a >= 0.0, a, 0.2 * a)
                wb[p][e, :] = jnp.exp(a)
                return 0

            def mpass(e, _):
                w = wb[p][e, :]
                for j in range(8):
                    hg[p][e, pl.ds(j * 16, 16)] = (
                        hg[p][e, pl.ds(j * 16, 16)] * w[head_of[j]])
                return 0

            lax.fori_loop(0, CH, wpass, 0)
            lax.fori_loop(0, CH, mpass, 0, unroll=2)
            fire_scatters(p, b)

            @pl.when(g + 2 < CPT)
            def _prefetch_base():
                for cp in base_copies(g + 2, rf, p):
                    cp.start()

        def pipe(i, _):
            for b in range(4):
                half(4 * i + b, b)
            return 0

        lax.fori_loop(0, CPT // 4, pipe, 0)
        for cp in scatter_copies(0, 2) + scatter_copies(1, 3):
            cp.wait()
        plsc.subcore_barrier()

        pltpu.sync_copy(num_sh.at[pl.ds(r0, NPT)], num_out.at[pl.ds(c * N + r0, NPT)])
        pltpu.sync_copy(den_sh.at[pl.ds(r0, NPT)], den_out.at[pl.ds(c * N + r0, NPT)])

        @pl.when(s == 0)
        def _write_tail():
            pltpu.sync_copy(num_sh.at[pl.ds(NS * NPT, NTAIL)],
                            num_out.at[pl.ds(c * N + NS * NPT, NTAIL)])
            pltpu.sync_copy(den_sh.at[pl.ds(NS * NPT, NTAIL)],
                            den_out.at[pl.ds(c * N + NS * NPT, NTAIL)])

    return edge_pass


_edge_pass_h8 = _make_edge_pass(8)
_edge_pass_h1 = _make_edge_pass(1)


def _fold_we(We, a_e, H, C):
    wf = (We.reshape(We.shape[0], H, C) * a_e[None]).sum(-1)
    return jnp.pad(wf, ((0, 0), (0, 16 - H)))

def _a_mat(a, H, C):
    m = jnp.zeros((H * C, 16), jnp.float32)
    for h in range(H):
        m = m.at[h * C:(h + 1) * C, h].set(a[h])
    return m

def _expand_mat(H):
    m = np.zeros((16, 128), np.float32)
    C = 128 // H
    for h in range(H):
        m[h, h * C:(h + 1) * C] = 1.0
    return jnp.asarray(m)


def kernel(x, edge_index, edge_attr, W1, We1, as1, ad1, ae1, b1,
           W2, We2, as2, ad2, ae2, b2, W3, We3, as3, ad3, ae3, b3, Wl, bl):
    src = edge_index[0].reshape(E // CH, CH)
    dst = edge_index[1].reshape(E // CH, CH)

    eap = edge_attr.reshape(E // 8, 128)
    eye8 = jnp.eye(8, dtype=jnp.float32)
    ale1, ale2, ale3 = _edge_logits(
        eap, jnp.kron(eye8, _fold_we(We1, ae1, 8, 16)),
        jnp.kron(eye8, _fold_we(We2, ae2, 8, 16)),
        jnp.kron(eye8, _fold_we(We3, ae3, 1, 128)))
    ale1 = ale1.reshape(E // CH, CH, 16)
    ale2 = ale2.reshape(E // CH, CH, 16)
    ale3 = ale3.reshape(E // CH, CH, 16)

    ex8 = _expand_mat(8)
    ex1 = _expand_mat(1)

    h, als, ald = _node_dense(x, W1, _a_mat(as1, 8, 16), _a_mat(ad1, 8, 16))
    num, den = _edge_pass_h8(src, dst, h, als, ald, ale1)

    h, als, ald = _node_dense(
        None, W2, _a_mat(as2, 8, 16), _a_mat(ad2, 8, 16),
        prev=(num, den, ex8, b1.reshape(1, 128)))
    num, den = _edge_pass_h8(src, dst, h, als, ald, ale2)

    h, als, ald = _node_dense(
        None, W3, _a_mat(as3, 1, 128), _a_mat(ad3, 1, 128),
        prev=(num, den, ex8, b2.reshape(1, 128)))
    num, den = _edge_pass_h1(src, dst, h, als, ald, ale3)

    return _final_dense(num, den, ex1,
                        b3.reshape(1, 128), Wl, bl.reshape(1, 128))
